# baseline (speedup 1.0000x reference)
import functools

import jax
import jax.numpy as jnp
from jax import lax
from jax.experimental import pallas as pl
from jax.experimental.pallas import tpu as pltpu
from jax.experimental.pallas import tpu_sc as plsc

_info = plsc.get_sparse_core_info()
_NC = _info.num_cores
_NS = _info.num_subcores
_NW = _NC * _NS

_PGRP = 1


@functools.cache
def _build_gather(b, p, v, d):
    assert b % _NW == 0 and p % _PGRP == 0
    cols = b // _NW
    rows_per_tile = (-(-v // _NS) + 7) & ~7
    full_tiles = min(_NS - 1, v // rows_per_tile)
    rows_rem = v - full_tiles * rows_per_tile

    mesh = plsc.VectorSubcoreMesh(core_axis_name="c", subcore_axis_name="s")

    @functools.partial(
        pl.kernel,
        out_type=jax.ShapeDtypeStruct((p, b, d), jnp.float32),
        mesh=mesh,
        scratch_types=[
            pltpu.VMEM((p, 128), jnp.int32),
            pltpu.VMEM((p, cols, d), jnp.float32),
            pltpu.VMEM_SHARED((v, d), jnp.float32),
        ]
        + [pltpu.SemaphoreType.DMA] * (p + 2),
    )
    def emb_kernel(idx_hbm, table_hbm, out_hbm, idx_v, rows_v, table_sp,
                   *sems):
        sem_s = sems[p]
        sem_i = sems[p + 1]
        sid = lax.axis_index("s")
        wid = sid * _NC + lax.axis_index("c")
        c0 = wid * cols
        ic0 = (wid // (128 // cols)) * 128
        qoff = (wid % (128 // cols)) * cols
        idx_cp = pltpu.async_copy(idx_hbm.at[:, pl.ds(ic0, 128)], idx_v, sem_i)

        @pl.when(sid < full_tiles)
        def _():
            pltpu.sync_copy(
                table_hbm.at[pl.ds(sid * rows_per_tile, rows_per_tile)],
                table_sp.at[pl.ds(sid * rows_per_tile, rows_per_tile)],
            )
        if rows_rem:
            @pl.when(sid == full_tiles)
            def _():
                pltpu.sync_copy(
                    table_hbm.at[pl.ds(full_tiles * rows_per_tile, rows_rem)],
                    table_sp.at[pl.ds(full_tiles * rows_per_tile, rows_rem)],
                )
        idx_cp.wait()
        plsc.subcore_barrier()
        gathers = []
        for j in range(p):
            gathers.append(
                pltpu.async_copy(
                    table_sp.at[idx_v.at[j, pl.ds(qoff, cols)]],
                    rows_v.at[j],
                    sems[j],
                )
            )
        scatters = []
        for j0 in range(0, p, _PGRP):
            for j in range(j0, j0 + _PGRP):
                gathers[j].wait()
            scatters.append(
                pltpu.async_copy(
                    rows_v.at[pl.ds(j0, _PGRP)],
                    out_hbm.at[pl.ds(j0, _PGRP), pl.ds(c0, cols)],
                    sem_s,
                )
            )
        for cp in scatters:
            cp.wait()

    return emb_kernel


def kernel(x, W):
    b, p = x.shape
    v, d = W.shape
    out_t = _build_gather(b, p, v, d)(x.T.astype(jnp.int32), W)
    return out_t.transpose(1, 0, 2)


# speedup vs baseline: 1.0128x; 1.0128x over previous
import functools

import jax
import jax.numpy as jnp
from jax import lax
from jax.experimental import pallas as pl
from jax.experimental.pallas import tpu as pltpu
from jax.experimental.pallas import tpu_sc as plsc

_info = plsc.get_sparse_core_info()
_NC = _info.num_cores
_NS = _info.num_subcores
_NW = _NC * _NS

_PGRP = 2


@functools.cache
def _build_gather(b, p, v, d):
    assert b % _NW == 0 and p % _PGRP == 0
    cols = b // _NW
    rows_per_tile = (-(-v // _NS) + 7) & ~7
    full_tiles = min(_NS - 1, v // rows_per_tile)
    rows_rem = v - full_tiles * rows_per_tile

    mesh = plsc.VectorSubcoreMesh(core_axis_name="c", subcore_axis_name="s")

    @functools.partial(
        pl.kernel,
        out_type=jax.ShapeDtypeStruct((p, b, d), jnp.float32),
        mesh=mesh,
        scratch_types=[
            pltpu.VMEM((p, 128), jnp.int32),
            pltpu.VMEM((p, cols, d), jnp.float32),
            pltpu.VMEM_SHARED((v, d), jnp.float32),
        ]
        + [pltpu.SemaphoreType.DMA] * (p + 2),
    )
    def emb_kernel(idx_hbm, table_hbm, out_hbm, idx_v, rows_v, table_sp,
                   *sems):
        sem_s = sems[p]
        sem_i = sems[p + 1]
        sid = lax.axis_index("s")
        wid = sid * _NC + lax.axis_index("c")
        c0 = wid * cols
        ic0 = (wid // (128 // cols)) * 128
        qoff = (wid % (128 // cols)) * cols
        idx_cp = pltpu.async_copy(idx_hbm.at[:, pl.ds(ic0, 128)], idx_v, sem_i)

        @pl.when(sid < full_tiles)
        def _():
            pltpu.sync_copy(
                table_hbm.at[pl.ds(sid * rows_per_tile, rows_per_tile)],
                table_sp.at[pl.ds(sid * rows_per_tile, rows_per_tile)],
            )
        if rows_rem:
            @pl.when(sid == full_tiles)
            def _():
                pltpu.sync_copy(
                    table_hbm.at[pl.ds(full_tiles * rows_per_tile, rows_rem)],
                    table_sp.at[pl.ds(full_tiles * rows_per_tile, rows_rem)],
                )
        idx_cp.wait()
        plsc.subcore_barrier()
        gathers = []
        for j in range(p):
            gathers.append(
                pltpu.async_copy(
                    table_sp.at[idx_v.at[j, pl.ds(qoff, cols)]],
                    rows_v.at[j],
                    sems[j],
                )
            )
        scatters = []
        for j0 in range(0, p, _PGRP):
            for j in range(j0, j0 + _PGRP):
                gathers[j].wait()
            scatters.append(
                pltpu.async_copy(
                    rows_v.at[pl.ds(j0, _PGRP)],
                    out_hbm.at[pl.ds(j0, _PGRP), pl.ds(c0, cols)],
                    sem_s,
                )
            )
        for cp in scatters:
            cp.wait()

    return emb_kernel


def kernel(x, W):
    b, p = x.shape
    v, d = W.shape
    out_t = _build_gather(b, p, v, d)(x.T.astype(jnp.int32), W)
    return out_t.transpose(1, 0, 2)
